# baseline (speedup 1.0000x reference)
import functools

import jax
import jax.numpy as jnp
from jax import lax
from jax.experimental import pallas as pl
from jax.experimental.pallas import tpu as pltpu
from jax.experimental.pallas import tpu_sc as plsc

VOCAB = 1000
BATCH = 16384
DIM = 1000
VPAD = 1024
D_PER_W = 32
BBLK = 1024
NBLK = BATCH // BBLK
GRP = BBLK // 16

_INFO = plsc.get_sparse_core_info()
NC = _INFO.num_cores
NS = _INFO.num_subcores
NW = NC * NS

_MESH = plsc.VectorSubcoreMesh(core_axis_name="c", subcore_axis_name="s")


@functools.partial(
    pl.kernel,
    mesh=_MESH,
    out_type=jax.ShapeDtypeStruct((DIM, BATCH), jnp.float32),
    scratch_types=[
        pltpu.VMEM((BATCH,), jnp.int32),
        pltpu.VMEM((D_PER_W * VPAD,), jnp.float32),
        pltpu.VMEM((D_PER_W, BBLK), jnp.float32),
        pltpu.VMEM((D_PER_W, BBLK), jnp.float32),
        pltpu.SemaphoreType.DMA,
        pltpu.SemaphoreType.DMA,
        pltpu.SemaphoreType.DMA,
    ],
    compiler_params=pltpu.CompilerParams(needs_layout_passes=False),
)
def _bgather(idx_hbm, tpad_hbm, out_hbm, idx_v, rows_v, ob0, ob1,
             sem0, sem1, sem_in):
    wid = lax.axis_index("s") * NC + lax.axis_index("c")
    d0 = jnp.minimum(wid * D_PER_W, DIM - D_PER_W)
    copies = [pltpu.async_copy(idx_hbm, idx_v, sem_in)]
    for r in range(D_PER_W):
        copies.append(pltpu.async_copy(
            tpad_hbm.at[d0 + r], rows_v.at[pl.ds(r * VPAD, VPAD)], sem_in))
    for c in copies:
        c.wait()

    def half_step(i, bb, ob, sem):
        @pl.when(i > 0)
        def _wait_prev():
            pltpu.make_async_copy(
                ob, out_hbm.at[pl.ds(d0, D_PER_W),
                               pl.ds((bb - 2) * BBLK, BBLK)], sem).wait()

        base = bb * BBLK

        @plsc.parallel_loop(0, GRP, unroll=4)
        def g_body(g):
            iv = idx_v[pl.ds(base + g * 16, 16)]
            for d in range(D_PER_W):
                val = plsc.load_gather(rows_v, [iv + (d * VPAD)])
                ob[d, pl.ds(g * 16, 16)] = val

        pltpu.async_copy(
            ob, out_hbm.at[pl.ds(d0, D_PER_W), pl.ds(base, BBLK)], sem)

    def bb_body(i, carry):
        half_step(i, 2 * i, ob0, sem0)
        half_step(i, 2 * i + 1, ob1, sem1)
        return carry

    lax.fori_loop(0, NBLK // 2, bb_body, 0)
    pltpu.make_async_copy(
        ob0, out_hbm.at[pl.ds(d0, D_PER_W),
                        pl.ds((NBLK - 2) * BBLK, BBLK)], sem0).wait()
    pltpu.make_async_copy(
        ob1, out_hbm.at[pl.ds(d0, D_PER_W),
                        pl.ds((NBLK - 1) * BBLK, BBLK)], sem1).wait()


def kernel(idx, table):
    tpad = jnp.pad(table, ((0, VPAD - VOCAB), (0, 0))).T
    out_t = _bgather(idx.astype(jnp.int32), tpad)
    return out_t.T


# speedup vs baseline: 1.2380x; 1.2380x over previous
import functools

import jax
import jax.numpy as jnp
from jax import lax
from jax.experimental import pallas as pl
from jax.experimental.pallas import tpu as pltpu
from jax.experimental.pallas import tpu_sc as plsc

VOCAB = 1000
BATCH = 16384
DIM = 1000
VPAD = 1024

_INFO = plsc.get_sparse_core_info()
NC = _INFO.num_cores
NS = _INFO.num_subcores
NW = NC * NS
D_PER_W = 32
BBLK = 1024
NBLK = BATCH // BBLK
GRP = BBLK // 16

_MESH = plsc.VectorSubcoreMesh(core_axis_name="c", subcore_axis_name="s")


@functools.partial(
    pl.kernel,
    mesh=_MESH,
    out_type=jax.ShapeDtypeStruct((DIM, BATCH), jnp.float32),
    scratch_types=[
        pltpu.VMEM((BATCH,), jnp.int32),
        pltpu.VMEM((D_PER_W * VPAD,), jnp.float32),
        pltpu.VMEM((D_PER_W, BBLK), jnp.float32),
        pltpu.VMEM((D_PER_W, BBLK), jnp.float32),
        pltpu.SemaphoreType.DMA,
        pltpu.SemaphoreType.DMA,
    ],
    compiler_params=pltpu.CompilerParams(needs_layout_passes=False),
)
def _bgather(idx_hbm, tflat_hbm, out_hbm, idx_v, rows_v, ob0, ob1,
             sem0, sem1):
    wid = lax.axis_index("s") * NC + lax.axis_index("c")
    d0 = jnp.minimum(wid * D_PER_W, DIM - D_PER_W)
    copies = [pltpu.async_copy(idx_hbm, idx_v, sem0)]
    for r in range(D_PER_W):
        copies.append(pltpu.async_copy(
            tflat_hbm.at[d0 + r], rows_v.at[pl.ds(r * VPAD, VPAD)], sem0))
    for c in copies:
        c.wait()

    obs = (ob0, ob1)
    sems = (sem0, sem1)
    pending = [None, None]

    for bb in range(NBLK):
        p = bb % 2
        if pending[p] is not None:
            pending[p].wait()
            pending[p] = None
        ob = obs[p]

        @plsc.parallel_loop(0, GRP, unroll=4)
        def g_body(g, _ob=ob, _bb=bb):
            iv = idx_v[pl.ds(_bb * BBLK + g * 16, 16)]
            for d in range(D_PER_W):
                val = plsc.load_gather(rows_v, [iv + (d * VPAD)])
                _ob[d, pl.ds(g * 16, 16)] = val
        pending[p] = pltpu.async_copy(
            ob, out_hbm.at[pl.ds(d0, D_PER_W), pl.ds(bb * BBLK, BBLK)],
            sems[p])

    for p in pending:
        if p is not None:
            p.wait()


def kernel(idx, table):
    tflat = jnp.pad(table, ((0, VPAD - VOCAB), (0, 0))).T
    out_t = _bgather(idx.astype(jnp.int32), tflat)
    return out_t.T
